# Initial kernel scaffold; baseline (speedup 1.0000x reference)
import jax
import jax.numpy as jnp
from jax.experimental import pallas as pl


def kernel(xyz, features):
    raise NotImplementedError("write your pallas kernel here")


# speedup vs baseline: 6.6534x; 6.6534x over previous
import functools

import numpy as np
import jax
import jax.numpy as jnp
from jax import lax
from jax.experimental import pallas as pl
from jax.experimental.pallas import tpu as pltpu
from jax.experimental.pallas import tpu_sc as plsc

B, N, C, S, K = 4, 4096, 256, 512, 32
RAD2 = np.float32(0.2 ** 2)
SUB = 8
LN = N // SUB


def _fps_body(xyz_v_ref, xyz_s_ref, nxyz_ref):
    x0 = xyz_v_ref[0, 0]
    x1 = xyz_v_ref[0, 1]
    x2 = xyz_v_ref[0, 2]
    row = lax.broadcasted_iota(jnp.int32, (SUB, LN), 0)
    col = lax.broadcasted_iota(jnp.int32, (SUB, LN), 1)
    iota_n = row * LN + col

    def step(i, carry):
        dist, f = carry
        c0 = xyz_s_ref[0, 0, f]
        c1 = xyz_s_ref[0, 1, f]
        c2 = xyz_s_ref[0, 2, f]
        nxyz_ref[0, 0, i] = c0
        nxyz_ref[0, 1, i] = c1
        nxyz_ref[0, 2, i] = c2
        d = (x0 - c0) ** 2 + (x1 - c1) ** 2 + (x2 - c2) ** 2
        dist = jnp.minimum(dist, d)
        m = jnp.max(dist)
        f2 = jnp.min(jnp.where(dist == m, iota_n, N)).astype(jnp.int32)
        return dist, f2

    dist0 = jnp.full((SUB, LN), 1e10, dtype=jnp.float32)
    lax.fori_loop(0, S, step, (dist0, jnp.int32(0)))


def _mask_body(nx_ref, xz_ref, m_ref):
    s = nx_ref[0]
    x = xz_ref[0]
    dot = lax.dot_general(
        s.astype(jnp.bfloat16), x.astype(jnp.bfloat16),
        dimension_numbers=(((1,), (0,)), ((), ())),
        preferred_element_type=jnp.float32,
    )
    t = (-2.0) * dot
    t = t + jnp.sum(s * s, axis=1, keepdims=True)
    t = t + jnp.sum(x * x, axis=0, keepdims=True)
    m_ref[0] = (t <= RAD2).astype(jnp.int32)


def _make_pool_kernel(rows_per_w, n_chunks):
    info = plsc.get_sparse_core_info()
    nc = info.num_cores
    mesh = plsc.VectorSubcoreMesh(core_axis_name="c", subcore_axis_name="s")

    @functools.partial(
        pl.kernel,
        out_type=jax.ShapeDtypeStruct((B * S, C), jnp.float32),
        mesh=mesh,
        scratch_types=[
            pltpu.VMEM((N,), jnp.int32),
            pltpu.VMEM((K,), jnp.int32),
            pltpu.VMEM((K, C), jnp.float32),
            pltpu.VMEM((C,), jnp.float32),
            pltpu.SemaphoreType.DMA,
        ],
        compiler_params=pltpu.CompilerParams(needs_layout_passes=False),
    )
    def pool(mask_hbm, feat_hbm, out_hbm, mrow, idxb, rows, orow, sem):
        wid = lax.axis_index("s") * nc + lax.axis_index("c")
        gbase = (wid // (S // rows_per_w)) * N
        iota = lax.iota(jnp.int32, 16)
        zeros16 = jnp.zeros((16,), jnp.int32)

        def row_body(k, _):
            r = wid * rows_per_w + k
            pltpu.sync_copy(mask_hbm.at[r], mrow)

            def sbody(t, cur):
                ids = t * 16 + iota
                m = plsc.load_gather(mrow, [ids])
                cs = plsc.cumsum(m)
                sel = jnp.logical_and(m > 0, (cur + cs) <= K)
                pos = jnp.where(sel, cur + cs - 1, 0)
                plsc.store_scatter(idxb, [pos], gbase + ids, mask=sel)
                return cur + jnp.sum(m)

            cur = lax.fori_loop(0, n_chunks, sbody, jnp.int32(0))
            cnt = jnp.minimum(cur, K)
            first = plsc.load_gather(idxb, [zeros16])
            for h in range(0, K, 16):
                curv = idxb[pl.ds(h, 16)]
                slot = h + iota
                idxb[pl.ds(h, 16)] = jnp.where(slot < cnt, curv, first)

            pltpu.async_copy(feat_hbm.at[idxb], rows, sem).wait()
            for cc in range(0, C, 16):
                acc = rows[0, pl.ds(cc, 16)]
                for j in range(1, K):
                    acc = jnp.maximum(acc, rows[j, pl.ds(cc, 16)])
                orow[pl.ds(cc, 16)] = acc
            pltpu.sync_copy(orow, out_hbm.at[r])
            return 0

        lax.fori_loop(0, rows_per_w, row_body, 0)

    return pool


@jax.jit
def kernel(xyz, features):
    xyz_t = jnp.transpose(xyz, (0, 2, 1))
    xyz_v = xyz_t.reshape(B, 3, SUB, LN)

    nxyz_t = pl.pallas_call(
        _fps_body,
        grid=(B,),
        in_specs=[
            pl.BlockSpec((1, 3, SUB, LN), lambda b: (b, 0, 0, 0)),
            pl.BlockSpec((1, 3, N), lambda b: (b, 0, 0), memory_space=pltpu.SMEM),
        ],
        out_specs=pl.BlockSpec((1, 3, S), lambda b: (b, 0, 0), memory_space=pltpu.SMEM),
        out_shape=jax.ShapeDtypeStruct((B, 3, S), jnp.float32),
    )(xyz_v, xyz_t)
    new_xyz = jnp.transpose(nxyz_t, (0, 2, 1))

    mask = pl.pallas_call(
        _mask_body,
        grid=(B,),
        in_specs=[
            pl.BlockSpec((1, S, 3), lambda b: (b, 0, 0)),
            pl.BlockSpec((1, 3, N), lambda b: (b, 0, 0)),
        ],
        out_specs=pl.BlockSpec((1, S, N), lambda b: (b, 0, 0)),
        out_shape=jax.ShapeDtypeStruct((B, S, N), jnp.int32),
    )(new_xyz, xyz_t)

    feat_rows = jnp.transpose(features, (0, 2, 1)).reshape(B * N, C)

    info = plsc.get_sparse_core_info()
    nw = info.num_cores * info.num_subcores
    pool = _make_pool_kernel((B * S) // nw, N // 16)
    pooled = pool(mask.reshape(B * S, N), feat_rows)

    sub_features = jnp.transpose(pooled.reshape(B, S, C), (0, 2, 1))
    return new_xyz, sub_features


# speedup vs baseline: 8.1203x; 1.2205x over previous
import functools

import numpy as np
import jax
import jax.numpy as jnp
from jax import lax
from jax.experimental import pallas as pl
from jax.experimental.pallas import tpu as pltpu
from jax.experimental.pallas import tpu_sc as plsc

B, N, C, S, K = 4, 4096, 256, 512, 32
RAD2 = np.float32(0.2 ** 2)
SUB = 8
LN = N // SUB


def _fps_body(xyz_v_ref, xyz_s_ref, nxyz_ref):
    xs = [[xyz_v_ref[b, d] for d in range(3)] for b in range(B)]
    row = lax.broadcasted_iota(jnp.int32, (SUB, LN), 0)
    col = lax.broadcasted_iota(jnp.int32, (SUB, LN), 1)
    iota_n = row * LN + col

    def step(i, carry):
        dists, fs = carry
        new_dists, new_fs = [], []
        for b in range(B):
            f = fs[b]
            c0 = xyz_s_ref[b, 0, f]
            c1 = xyz_s_ref[b, 1, f]
            c2 = xyz_s_ref[b, 2, f]
            nxyz_ref[b, 0, i] = c0
            nxyz_ref[b, 1, i] = c1
            nxyz_ref[b, 2, i] = c2
            x0, x1, x2 = xs[b]
            d = (x0 - c0) ** 2 + (x1 - c1) ** 2 + (x2 - c2) ** 2
            dist = jnp.minimum(dists[b], d)
            m = jnp.max(dist)
            f2 = jnp.min(jnp.where(dist == m, iota_n, N)).astype(jnp.int32)
            new_dists.append(dist)
            new_fs.append(f2)
        return tuple(new_dists), tuple(new_fs)

    dist0 = jnp.full((SUB, LN), 1e10, dtype=jnp.float32)
    lax.fori_loop(0, S, step,
                  (tuple(dist0 for _ in range(B)),
                   tuple(jnp.int32(0) for _ in range(B))))


def _mask_body(nx_ref, xz_ref, m_ref):
    s = nx_ref[0]
    x = xz_ref[0]
    dot = lax.dot_general(
        s.astype(jnp.bfloat16), x.astype(jnp.bfloat16),
        dimension_numbers=(((1,), (0,)), ((), ())),
        preferred_element_type=jnp.float32,
    )
    t = (-2.0) * dot
    t = t + jnp.sum(s * s, axis=1, keepdims=True)
    t = t + jnp.sum(x * x, axis=0, keepdims=True)
    m_ref[0] = (t <= RAD2).astype(jnp.int32)


def _make_pool_kernel(rows_per_w, n_chunks):
    info = plsc.get_sparse_core_info()
    nc = info.num_cores
    mesh = plsc.VectorSubcoreMesh(core_axis_name="c", subcore_axis_name="s")

    @functools.partial(
        pl.kernel,
        out_type=jax.ShapeDtypeStruct((B * S, C), jnp.float32),
        mesh=mesh,
        scratch_types=[
            pltpu.VMEM((N,), jnp.int32),
            pltpu.VMEM((K,), jnp.int32),
            pltpu.VMEM((K, C), jnp.float32),
            pltpu.VMEM((C,), jnp.float32),
            pltpu.SemaphoreType.DMA,
        ],
        compiler_params=pltpu.CompilerParams(needs_layout_passes=False),
    )
    def pool(mask_hbm, feat_hbm, out_hbm, mrow, idxb, rows, orow, sem):
        wid = lax.axis_index("s") * nc + lax.axis_index("c")
        gbase = (wid // (S // rows_per_w)) * N
        iota = lax.iota(jnp.int32, 16)
        zeros16 = jnp.zeros((16,), jnp.int32)

        def row_body(k, _):
            r = wid * rows_per_w + k
            pltpu.sync_copy(mask_hbm.at[r], mrow)

            def sbody(t, cur):
                ids = t * 16 + iota
                m = plsc.load_gather(mrow, [ids])
                cs = plsc.cumsum(m)
                sel = jnp.logical_and(m > 0, (cur + cs) <= K)
                pos = jnp.where(sel, cur + cs - 1, 0)
                plsc.store_scatter(idxb, [pos], gbase + ids, mask=sel)
                return cur + jnp.sum(m)

            cur = lax.fori_loop(0, n_chunks, sbody, jnp.int32(0))
            cnt = jnp.minimum(cur, K)
            first = plsc.load_gather(idxb, [zeros16])
            for h in range(0, K, 16):
                curv = idxb[pl.ds(h, 16)]
                slot = h + iota
                idxb[pl.ds(h, 16)] = jnp.where(slot < cnt, curv, first)

            pltpu.async_copy(feat_hbm.at[idxb], rows, sem).wait()
            for cc in range(0, C, 16):
                acc = rows[0, pl.ds(cc, 16)]
                for j in range(1, K):
                    acc = jnp.maximum(acc, rows[j, pl.ds(cc, 16)])
                orow[pl.ds(cc, 16)] = acc
            pltpu.sync_copy(orow, out_hbm.at[r])
            return 0

        lax.fori_loop(0, rows_per_w, row_body, 0)

    return pool


@jax.jit
def kernel(xyz, features):
    xyz_t = jnp.transpose(xyz, (0, 2, 1))
    xyz_v = xyz_t.reshape(B, 3, SUB, LN)

    nxyz_t = pl.pallas_call(
        _fps_body,
        in_specs=[
            pl.BlockSpec((B, 3, SUB, LN), lambda: (0, 0, 0, 0)),
            pl.BlockSpec((B, 3, N), lambda: (0, 0, 0), memory_space=pltpu.SMEM),
        ],
        out_specs=pl.BlockSpec((B, 3, S), lambda: (0, 0, 0), memory_space=pltpu.SMEM),
        out_shape=jax.ShapeDtypeStruct((B, 3, S), jnp.float32),
    )(xyz_v, xyz_t)
    new_xyz = jnp.transpose(nxyz_t, (0, 2, 1))

    mask = pl.pallas_call(
        _mask_body,
        grid=(B,),
        in_specs=[
            pl.BlockSpec((1, S, 3), lambda b: (b, 0, 0)),
            pl.BlockSpec((1, 3, N), lambda b: (b, 0, 0)),
        ],
        out_specs=pl.BlockSpec((1, S, N), lambda b: (b, 0, 0)),
        out_shape=jax.ShapeDtypeStruct((B, S, N), jnp.int32),
    )(new_xyz, xyz_t)

    feat_rows = jnp.transpose(features, (0, 2, 1)).reshape(B * N, C)

    info = plsc.get_sparse_core_info()
    nw = info.num_cores * info.num_subcores
    pool = _make_pool_kernel((B * S) // nw, N // 16)
    pooled = pool(mask.reshape(B * S, N), feat_rows)

    sub_features = jnp.transpose(pooled.reshape(B, S, C), (0, 2, 1))
    return new_xyz, sub_features


# speedup vs baseline: 8.6740x; 1.0682x over previous
import functools

import numpy as np
import jax
import jax.numpy as jnp
from jax import lax
from jax.experimental import pallas as pl
from jax.experimental.pallas import tpu as pltpu
from jax.experimental.pallas import tpu_sc as plsc

B, N, C, S, K = 4, 4096, 256, 512, 32
RAD2 = np.float32(0.2 ** 2)
SUB = 8
LN = N // SUB


def _fps_body(xyz_v_ref, xyz_s_ref, nxyz_ref):
    xs = [[xyz_v_ref[b, d] for d in range(3)] for b in range(B)]
    row = lax.broadcasted_iota(jnp.int32, (SUB, LN), 0)
    col = lax.broadcasted_iota(jnp.int32, (SUB, LN), 1)
    iota_n = row * LN + col

    def step(i, carry):
        dists, fs = carry
        new_dists, new_fs = [], []
        for b in range(B):
            f = fs[b]
            c0 = xyz_s_ref[b, 0, f]
            c1 = xyz_s_ref[b, 1, f]
            c2 = xyz_s_ref[b, 2, f]
            nxyz_ref[b, 0, i] = c0
            nxyz_ref[b, 1, i] = c1
            nxyz_ref[b, 2, i] = c2
            x0, x1, x2 = xs[b]
            d = (x0 - c0) ** 2 + (x1 - c1) ** 2 + (x2 - c2) ** 2
            dist = jnp.minimum(dists[b], d)
            m = jnp.max(dist)
            f2 = jnp.min(jnp.where(dist == m, iota_n, N)).astype(jnp.int32)
            new_dists.append(dist)
            new_fs.append(f2)
        return tuple(new_dists), tuple(new_fs)

    dist0 = jnp.full((SUB, LN), 1e10, dtype=jnp.float32)
    lax.fori_loop(0, S, step,
                  (tuple(dist0 for _ in range(B)),
                   tuple(jnp.int32(0) for _ in range(B))))


def _mask_body(nx_ref, xz_ref, m_ref, nsc_ref):
    s = nx_ref[0]
    x = xz_ref[0]
    dot = lax.dot_general(
        s.astype(jnp.bfloat16), x.astype(jnp.bfloat16),
        dimension_numbers=(((1,), (0,)), ((), ())),
        preferred_element_type=jnp.float32,
    )
    t = (-2.0) * dot
    t = t + jnp.sum(s * s, axis=1, keepdims=True)
    t = t + jnp.sum(x * x, axis=0, keepdims=True)
    mi = (t <= RAD2).astype(jnp.int32)
    m_ref[0] = mi
    c = mi
    sh = 1
    while sh < N:
        c = c + jnp.concatenate(
            [jnp.zeros((S, sh), jnp.int32), c[:, : N - sh]], axis=1)
        sh *= 2
    l = jnp.sum((c < K).astype(jnp.int32), axis=1)
    nsc_ref[0] = jnp.minimum(l // 16 + 1, N // 16).reshape(1, S)


def _make_pool_kernel(rows_per_w, n_chunks):
    info = plsc.get_sparse_core_info()
    nc = info.num_cores
    mesh = plsc.VectorSubcoreMesh(core_axis_name="c", subcore_axis_name="s")

    @functools.partial(
        pl.kernel,
        out_type=jax.ShapeDtypeStruct((B * S, C), jnp.float32),
        mesh=mesh,
        scratch_types=[
            pltpu.VMEM((N,), jnp.int32),
            pltpu.VMEM((K,), jnp.int32),
            pltpu.VMEM((K, C), jnp.float32),
            pltpu.VMEM((C,), jnp.float32),
            pltpu.VMEM((rows_per_w,), jnp.int32),
            pltpu.SemaphoreType.DMA,
        ],
        compiler_params=pltpu.CompilerParams(needs_layout_passes=False),
    )
    def pool(mask_hbm, feat_hbm, nscan_hbm, out_hbm, mrow, idxb, rows, orow,
             nscv, sem):
        wid = lax.axis_index("s") * nc + lax.axis_index("c")
        gbase = (wid // (S // rows_per_w)) * N
        iota = lax.iota(jnp.int32, 16)
        zeros16 = jnp.zeros((16,), jnp.int32)
        pltpu.sync_copy(nscan_hbm.at[pl.ds(wid * rows_per_w, rows_per_w)], nscv)

        def row_body(k, _):
            r = wid * rows_per_w + k
            pltpu.sync_copy(mask_hbm.at[r], mrow)
            nsc = jnp.max(plsc.load_gather(nscv, [jnp.full((16,), k, jnp.int32)]))

            def sbody(t, cur):
                ids = t * 16 + iota
                m = plsc.load_gather(mrow, [ids])
                cs = plsc.cumsum(m)
                sel = jnp.logical_and(m > 0, (cur + cs) <= K)
                pos = jnp.where(sel, cur + cs - 1, 0)
                plsc.store_scatter(idxb, [pos], gbase + ids, mask=sel)
                return cur + jnp.sum(m)

            cur = lax.fori_loop(0, nsc, sbody, jnp.int32(0))
            cnt = jnp.minimum(cur, K)
            first = plsc.load_gather(idxb, [zeros16])
            for h in range(0, K, 16):
                curv = idxb[pl.ds(h, 16)]
                slot = h + iota
                idxb[pl.ds(h, 16)] = jnp.where(slot < cnt, curv, first)

            pltpu.async_copy(feat_hbm.at[idxb], rows, sem).wait()
            for cc in range(0, C, 16):
                acc = rows[0, pl.ds(cc, 16)]
                for j in range(1, K):
                    acc = jnp.maximum(acc, rows[j, pl.ds(cc, 16)])
                orow[pl.ds(cc, 16)] = acc
            pltpu.sync_copy(orow, out_hbm.at[r])
            return 0

        lax.fori_loop(0, rows_per_w, row_body, 0)

    return pool


@jax.jit
def kernel(xyz, features):
    xyz_t = jnp.transpose(xyz, (0, 2, 1))
    xyz_v = xyz_t.reshape(B, 3, SUB, LN)

    nxyz_t = pl.pallas_call(
        _fps_body,
        in_specs=[
            pl.BlockSpec((B, 3, SUB, LN), lambda: (0, 0, 0, 0)),
            pl.BlockSpec((B, 3, N), lambda: (0, 0, 0), memory_space=pltpu.SMEM),
        ],
        out_specs=pl.BlockSpec((B, 3, S), lambda: (0, 0, 0), memory_space=pltpu.SMEM),
        out_shape=jax.ShapeDtypeStruct((B, 3, S), jnp.float32),
    )(xyz_v, xyz_t)
    new_xyz = jnp.transpose(nxyz_t, (0, 2, 1))

    mask, nscan = pl.pallas_call(
        _mask_body,
        grid=(B,),
        in_specs=[
            pl.BlockSpec((1, S, 3), lambda b: (b, 0, 0)),
            pl.BlockSpec((1, 3, N), lambda b: (b, 0, 0)),
        ],
        out_specs=[
            pl.BlockSpec((1, S, N), lambda b: (b, 0, 0)),
            pl.BlockSpec((1, 1, S), lambda b: (b, 0, 0)),
        ],
        out_shape=[
            jax.ShapeDtypeStruct((B, S, N), jnp.int32),
            jax.ShapeDtypeStruct((B, 1, S), jnp.int32),
        ],
    )(new_xyz, xyz_t)

    feat_rows = jnp.transpose(features, (0, 2, 1)).reshape(B * N, C)

    info = plsc.get_sparse_core_info()
    nw = info.num_cores * info.num_subcores
    pool = _make_pool_kernel((B * S) // nw, N // 16)
    pooled = pool(mask.reshape(B * S, N), feat_rows,
                  nscan.reshape(B * S))

    sub_features = jnp.transpose(pooled.reshape(B, S, C), (0, 2, 1))
    return new_xyz, sub_features


# speedup vs baseline: 13.7854x; 1.5893x over previous
import functools

import numpy as np
import jax
import jax.numpy as jnp
from jax import lax
from jax.experimental import pallas as pl
from jax.experimental.pallas import tpu as pltpu
from jax.experimental.pallas import tpu_sc as plsc

B, N, C, S, K = 4, 4096, 256, 512, 32
RAD2 = np.float32(0.2 ** 2)
SUB = 8
LN = N // SUB


def _fps_body(xyz_v_ref, xyz_s_ref, nxyz_ref):
    pass

    def step(i, carry):
        dists, fs = carry
        new_dists, new_fs = [], []
        for b in range(B):
            f = fs[b]
            c0 = xyz_s_ref[b, 0, f]
            c1 = xyz_s_ref[b, 1, f]
            c2 = xyz_s_ref[b, 2, f]
            nxyz_ref[b, 0, i] = c0
            nxyz_ref[b, 1, i] = c1
            nxyz_ref[b, 2, i] = c2
            x0, x1, x2 = (xyz_v_ref[b, d] for d in range(3))
            d = (x0 - c0) ** 2 + (x1 - c1) ** 2 + (x2 - c2) ** 2
            dist = jnp.minimum(dists[b], d)
            f2 = jnp.argmax(dist).astype(jnp.int32)
            new_dists.append(dist)
            new_fs.append(f2)
        return tuple(new_dists), tuple(new_fs)

    dist0 = jnp.full((SUB, LN), 1e10, dtype=jnp.float32)
    lax.fori_loop(0, S, step,
                  (tuple(dist0 for _ in range(B)),
                   tuple(jnp.int32(0) for _ in range(B))))


def _mask_body(nx_ref, xz_ref, m_ref, nsc_ref):
    s = nx_ref[0]
    x = xz_ref[0]
    dot = lax.dot_general(
        s.astype(jnp.bfloat16), x.astype(jnp.bfloat16),
        dimension_numbers=(((1,), (0,)), ((), ())),
        preferred_element_type=jnp.float32,
    )
    t = (-2.0) * dot
    t = t + jnp.sum(s * s, axis=1, keepdims=True)
    t = t + jnp.sum(x * x, axis=0, keepdims=True)
    mi = (t <= RAD2).astype(jnp.int32)
    m_ref[0] = mi
    c = mi
    sh = 1
    while sh < N:
        c = c + jnp.concatenate(
            [jnp.zeros((S, sh), jnp.int32), c[:, : N - sh]], axis=1)
        sh *= 2
    l = jnp.sum((c < K).astype(jnp.int32), axis=1)
    nsc_ref[0] = jnp.minimum(l // 16 + 1, N // 16).reshape(1, S)


def _make_pool_kernel(rows_per_w, n_chunks):
    info = plsc.get_sparse_core_info()
    nc = info.num_cores
    mesh = plsc.VectorSubcoreMesh(core_axis_name="c", subcore_axis_name="s")

    @functools.partial(
        pl.kernel,
        out_type=jax.ShapeDtypeStruct((B * S, C), jnp.float32),
        mesh=mesh,
        scratch_types=[
            pltpu.VMEM((N,), jnp.int32),
            pltpu.VMEM((K,), jnp.int32),
            pltpu.VMEM((K, C), jnp.float32),
            pltpu.VMEM((C,), jnp.float32),
            pltpu.VMEM((rows_per_w,), jnp.int32),
            pltpu.SemaphoreType.DMA,
        ],
        compiler_params=pltpu.CompilerParams(needs_layout_passes=False),
    )
    def pool(mask_hbm, feat_hbm, nscan_hbm, out_hbm, mrow, idxb, rows, orow,
             nscv, sem):
        wid = lax.axis_index("s") * nc + lax.axis_index("c")
        gbase = (wid // (S // rows_per_w)) * N
        iota = lax.iota(jnp.int32, 16)
        zeros16 = jnp.zeros((16,), jnp.int32)
        pltpu.sync_copy(nscan_hbm.at[pl.ds(wid * rows_per_w, rows_per_w)], nscv)

        def row_body(k, _):
            r = wid * rows_per_w + k
            pltpu.sync_copy(mask_hbm.at[r], mrow)
            nsc = jnp.max(plsc.load_gather(nscv, [jnp.full((16,), k, jnp.int32)]))

            def sbody(t, cur):
                ids = t * 16 + iota
                m = plsc.load_gather(mrow, [ids])
                cs = plsc.cumsum(m)
                sel = jnp.logical_and(m > 0, (cur + cs) <= K)
                pos = jnp.where(sel, cur + cs - 1, 0)
                plsc.store_scatter(idxb, [pos], gbase + ids, mask=sel)
                return cur + jnp.sum(m)

            cur = lax.fori_loop(0, nsc, sbody, jnp.int32(0))
            cnt = jnp.minimum(cur, K)
            first = plsc.load_gather(idxb, [zeros16])
            for h in range(0, K, 16):
                curv = idxb[pl.ds(h, 16)]
                slot = h + iota
                idxb[pl.ds(h, 16)] = jnp.where(slot < cnt, curv, first)

            pltpu.async_copy(feat_hbm.at[idxb], rows, sem).wait()
            for cc in range(0, C, 16):
                acc = rows[0, pl.ds(cc, 16)]
                for j in range(1, K):
                    acc = jnp.maximum(acc, rows[j, pl.ds(cc, 16)])
                orow[pl.ds(cc, 16)] = acc
            pltpu.sync_copy(orow, out_hbm.at[r])
            return 0

        lax.fori_loop(0, rows_per_w, row_body, 0)

    return pool


@jax.jit
def kernel(xyz, features):
    xyz_t = jnp.transpose(xyz, (0, 2, 1))
    xyz_v = xyz_t.reshape(B, 3, SUB, LN)

    nxyz_t = pl.pallas_call(
        _fps_body,
        in_specs=[
            pl.BlockSpec((B, 3, SUB, LN), lambda: (0, 0, 0, 0)),
            pl.BlockSpec((B, 3, N), lambda: (0, 0, 0), memory_space=pltpu.SMEM),
        ],
        out_specs=pl.BlockSpec((B, 3, S), lambda: (0, 0, 0), memory_space=pltpu.SMEM),
        out_shape=jax.ShapeDtypeStruct((B, 3, S), jnp.float32),
    )(xyz_v, xyz_t)
    new_xyz = jnp.transpose(nxyz_t, (0, 2, 1))

    mask, nscan = pl.pallas_call(
        _mask_body,
        grid=(B,),
        in_specs=[
            pl.BlockSpec((1, S, 3), lambda b: (b, 0, 0)),
            pl.BlockSpec((1, 3, N), lambda b: (b, 0, 0)),
        ],
        out_specs=[
            pl.BlockSpec((1, S, N), lambda b: (b, 0, 0)),
            pl.BlockSpec((1, 1, S), lambda b: (b, 0, 0)),
        ],
        out_shape=[
            jax.ShapeDtypeStruct((B, S, N), jnp.int32),
            jax.ShapeDtypeStruct((B, 1, S), jnp.int32),
        ],
    )(new_xyz, xyz_t)

    feat_rows = jnp.transpose(features, (0, 2, 1)).reshape(B * N, C)

    info = plsc.get_sparse_core_info()
    nw = info.num_cores * info.num_subcores
    pool = _make_pool_kernel((B * S) // nw, N // 16)
    pooled = pool(mask.reshape(B * S, N), feat_rows,
                  nscan.reshape(B * S))

    sub_features = jnp.transpose(pooled.reshape(B, S, C), (0, 2, 1))
    return new_xyz, sub_features
